# Initial kernel scaffold; baseline (speedup 1.0000x reference)
import jax
import jax.numpy as jnp
from jax.experimental import pallas as pl


def kernel(h, pos, edge_index, style, W1, b1, W2, b2, Wf, bf, Wg1, bg1, Wg2, bg2, Ws, bs):
    raise NotImplementedError("write your pallas kernel here")


# speedup vs baseline: 13.2021x; 13.2021x over previous
import functools

import jax
import jax.numpy as jnp
from jax import lax
from jax.experimental import pallas as pl
from jax.experimental.pallas import tpu as pltpu
from jax.experimental.pallas import tpu_sc as plsc

_NC = 2
_NS = 16
_NW = _NC * _NS
_CHUNK = 64
_LANES = 16


def _pre_body(h_ref, posp_ref, W1_ref, b1_ref, W2p_ref, b2p_ref, Wf3p_ref,
              WfH_ref, bf_ref, U_ref, V_ref):
    h = h_ref[...]
    posp = posp_ref[...]
    t = jnp.maximum(
        jnp.dot(h, W1_ref[...], preferred_element_type=jnp.float32) + b1_ref[...], 0.0)
    deltap = jnp.tanh(
        jnp.dot(t, W2p_ref[...], preferred_element_type=jnp.float32) + b2p_ref[...])
    U_ref[...] = (jnp.dot(h, WfH_ref[...], preferred_element_type=jnp.float32)
                  + jnp.dot(posp, Wf3p_ref[...], preferred_element_type=jnp.float32))
    V_ref[...] = (jnp.dot(deltap - posp, Wf3p_ref[...], preferred_element_type=jnp.float32)
                  + bf_ref[...])


def _post_body(p0_ref, p1_ref, h_ref, Wg1_ref, bg1_ref, Wg2_ref, bg2_ref,
               h2_ref, sum_ref, sumsq_ref):
    aggr = p0_ref[...] + p1_ref[...]
    t = jnp.maximum(
        jnp.dot(aggr, Wg1_ref[...], preferred_element_type=jnp.float32) + bg1_ref[...], 0.0)
    out = jnp.dot(t, Wg2_ref[...], preferred_element_type=jnp.float32) + bg2_ref[...]
    h2 = h_ref[...] + out
    h2 = jnp.where(h2 >= 0, h2, 0.2 * h2)
    h2_ref[...] = h2
    ps = jnp.sum(h2, axis=0, keepdims=True)
    pss = jnp.sum(h2 * h2, axis=0, keepdims=True)

    @pl.when(pl.program_id(0) == 0)
    def _():
        sum_ref[...] = ps
        sumsq_ref[...] = pss

    @pl.when(pl.program_id(0) != 0)
    def _():
        sum_ref[...] += ps
        sumsq_ref[...] += pss


def _make_final_body(n, c):
    def _final_body(h2_ref, style_ref, Ws_ref, bs_ref, sum_ref, sumsq_ref, o_ref):
        mean = sum_ref[...] / n
        var = sumsq_ref[...] / n - mean * mean
        rstd = lax.rsqrt(var + 1e-5)
        s = jnp.dot(style_ref[...], Ws_ref[...], preferred_element_type=jnp.float32) + bs_ref[...]
        gamma = s[:, :c]
        beta = s[:, c:]
        o_ref[...] = gamma * ((h2_ref[...] - mean) * rstd) + beta
    return _final_body


def _chunk_sizes(total):
    sizes = [_CHUNK] * (total // _CHUNK)
    if total % _CHUNK:
        sizes.append(total % _CHUNK)
    return sizes


def _make_edge_kernel(n, c, e):
    rows_main = ((n // 8) // _NS) * 8
    rows_last = n - (_NS - 1) * rows_main
    nchunks = e // _CHUNK
    iters = (nchunks + _NW - 1) // _NW
    nvec = c // _LANES
    mesh = plsc.VectorSubcoreMesh(core_axis_name="c", subcore_axis_name="s")

    @functools.partial(
        pl.kernel,
        out_type=jax.ShapeDtypeStruct((2 * n, c), jnp.float32),
        mesh=mesh,
        scratch_types=[
            pltpu.VMEM((4, _CHUNK), jnp.int32),
            pltpu.VMEM((4, _CHUNK), jnp.int32),
            pltpu.VMEM((2, _CHUNK, c), jnp.float32),
            pltpu.VMEM((2, _CHUNK, c), jnp.float32),
            pltpu.VMEM((2, _CHUNK, c), jnp.float32),
            pltpu.VMEM_SHARED((n, c), jnp.float32),
            pltpu.SemaphoreType.DMA((4,)),
            pltpu.SemaphoreType.DMA((2,)),
            pltpu.SemaphoreType.DMA((2,)),
        ],
    )
    def _edge(src_hbm, dst_hbm, u_hbm, v_hbm, out_hbm,
              sidx, didx, u_v, v_v, m_v, aggr_sh, semi, semg, semsc):
        ci = lax.axis_index("c")
        si = lax.axis_index("s")
        w = si * _NC + ci

        def cid_of(t):
            return w + t * _NW

        def issue_idx(t, ts):
            q = ts % 4

            @pl.when(cid_of(t) < nchunks)
            def _():
                off = cid_of(t) * _CHUNK
                pltpu.async_copy(src_hbm.at[pl.ds(off, _CHUNK)], sidx.at[q],
                                 semi.at[q])
                pltpu.async_copy(dst_hbm.at[pl.ds(off, _CHUNK)], didx.at[q],
                                 semi.at[q])

        def issue_gathers(t, ts):
            q, b = ts % 4, ts % 2

            @pl.when(cid_of(t) < nchunks)
            def _():
                off = cid_of(t) * _CHUNK
                pltpu.make_async_copy(src_hbm.at[pl.ds(off, _CHUNK)],
                                      sidx.at[q], semi.at[q]).wait()
                pltpu.make_async_copy(dst_hbm.at[pl.ds(off, _CHUNK)],
                                      didx.at[q], semi.at[q]).wait()
                pltpu.async_copy(u_hbm.at[sidx.at[q]], u_v.at[b], semg.at[b])
                pltpu.async_copy(v_hbm.at[didx.at[q]], v_v.at[b], semg.at[b])

        def wait_gathers(t, ts):
            q, b = ts % 4, ts % 2

            @pl.when(cid_of(t) < nchunks)
            def _():
                pltpu.make_async_copy(u_hbm.at[sidx.at[q]], u_v.at[b],
                                      semg.at[b]).wait()
                pltpu.make_async_copy(v_hbm.at[didx.at[q]], v_v.at[b],
                                      semg.at[b]).wait()

        def wait_scatter(ts):
            q, b = ts % 4, ts % 2
            pltpu.make_async_copy(m_v.at[b], aggr_sh.at[didx.at[q]],
                                  semsc.at[b]).wait()

        def drain_scatter(t, ts):
            @pl.when(cid_of(t) < nchunks)
            def _():
                wait_scatter(ts)

        def process(t, ts):
            q, b = ts % 4, ts % 2

            @pl.when(cid_of(t) < nchunks)
            def _():
                def crow(r, cc):
                    for j in range(nvec):
                        sl = pl.ds(j * _LANES, _LANES)
                        m_v[b, r, sl] = jnp.maximum(
                            u_v[b, r, sl] + v_v[b, r, sl], 0.0)
                    return cc
                lax.fori_loop(0, _CHUNK, crow, 0)
                pltpu.async_copy(m_v.at[b], aggr_sh.at[didx.at[q]],
                                 semsc.at[b], add=True)

        def zrow(r, carry):
            for j in range(nvec):
                m_v[0, r, pl.ds(j * _LANES, _LANES)] = jnp.zeros(
                    (_LANES,), jnp.float32)
            return carry
        lax.fori_loop(0, _CHUNK, zrow, 0)

        base = pl.multiple_of(si * rows_main, 8)

        def _stripe_copy(row_fn):
            @pl.when(si < _NS - 1)
            def _():
                off = 0
                for sz in _chunk_sizes(rows_main):
                    row_fn(off, sz)
                    off += sz

            @pl.when(si == _NS - 1)
            def _():
                off = 0
                for sz in _chunk_sizes(rows_last):
                    row_fn(off, sz)
                    off += sz

        _stripe_copy(lambda off, sz: pltpu.sync_copy(
            m_v.at[0, pl.ds(0, sz)], aggr_sh.at[pl.ds(base + off, sz)]))
        plsc.subcore_barrier()

        issue_idx(0, 0)
        issue_idx(1, 1)
        issue_gathers(0, 0)
        for t0 in (0, 1):
            wait_gathers(t0, t0)
            issue_gathers(t0 + 1, t0 + 1)
            process(t0, t0)
            issue_idx(t0 + 2, t0 + 2)

        nquads = (iters + 3) // 4

        def outer(g, carry):
            for ts in (2, 3, 4, 5):
                t = 4 * g + ts
                wait_gathers(t, ts)
                issue_gathers(t + 1, ts + 1)
                drain_scatter(t - 2, ts - 2)
                process(t, ts)
                issue_idx(t + 2, ts + 2)
            return carry
        lax.fori_loop(0, nquads, outer, 0)

        tlast = 2 + 4 * nquads
        drain_scatter(tlast - 2, tlast - 2)
        drain_scatter(tlast - 1, tlast - 1)
        plsc.subcore_barrier()

        obase = pl.multiple_of(ci * n + base, 8)
        _stripe_copy(lambda off, sz: pltpu.sync_copy(
            aggr_sh.at[pl.ds(base + off, sz)],
            out_hbm.at[pl.ds(obase + off, sz)]))

    return _edge


def kernel(h, pos, edge_index, style, W1, b1, W2, b2, Wf, bf, Wg1, bg1,
           Wg2, bg2, Ws, bs):
    n, c = h.shape
    e = edge_index.shape[1]
    csty = style.shape[1]
    blk = 2000
    grid = n // blk

    src = edge_index[0]
    dst = edge_index[1]
    posp = jnp.pad(pos, ((0, 0), (0, c - pos.shape[1])))
    W2p = jnp.zeros((c, c), jnp.float32).at[:, :pos.shape[1]].set(W2)
    b2p = jnp.zeros((1, c), jnp.float32).at[0, :pos.shape[1]].set(b2)
    Wf3p = jnp.zeros((c, c), jnp.float32).at[:pos.shape[1], :].set(Wf[:pos.shape[1]])
    WfH = Wf[pos.shape[1]:]

    row_spec = pl.BlockSpec((blk, c), lambda i: (i, 0))
    mat_spec = pl.BlockSpec((c, c), lambda i: (0, 0))
    vec_spec = pl.BlockSpec((1, c), lambda i: (0, 0))

    U, V = pl.pallas_call(
        _pre_body,
        grid=(grid,),
        in_specs=[row_spec, row_spec, mat_spec, vec_spec, mat_spec, vec_spec,
                  mat_spec, mat_spec, vec_spec],
        out_specs=[row_spec, row_spec],
        out_shape=[jax.ShapeDtypeStruct((n, c), jnp.float32)] * 2,
    )(h, posp, W1, b1.reshape(1, c), W2p, b2p, Wf3p, WfH, bf.reshape(1, c))

    parts = _make_edge_kernel(n, c, e)(src, dst, U, V)
    p0 = parts[:n]
    p1 = parts[n:]

    h2, sums, sumsq = pl.pallas_call(
        _post_body,
        grid=(grid,),
        in_specs=[row_spec, row_spec, row_spec, mat_spec, vec_spec, mat_spec,
                  vec_spec],
        out_specs=[row_spec, vec_spec, vec_spec],
        out_shape=[jax.ShapeDtypeStruct((n, c), jnp.float32),
                   jax.ShapeDtypeStruct((1, c), jnp.float32),
                   jax.ShapeDtypeStruct((1, c), jnp.float32)],
    )(p0, p1, h, Wg1, bg1.reshape(1, c), Wg2, bg2.reshape(1, c))

    out = pl.pallas_call(
        _make_final_body(n, c),
        grid=(grid,),
        in_specs=[row_spec,
                  pl.BlockSpec((blk, csty), lambda i: (i, 0)),
                  pl.BlockSpec((csty, 2 * c), lambda i: (0, 0)),
                  pl.BlockSpec((1, 2 * c), lambda i: (0, 0)),
                  vec_spec, vec_spec],
        out_specs=row_spec,
        out_shape=jax.ShapeDtypeStruct((n, c), jnp.float32),
    )(h2, style, Ws, bs.reshape(1, 2 * c), sums, sumsq)
    return out


# speedup vs baseline: 13.3154x; 1.0086x over previous
import functools

import jax
import jax.numpy as jnp
from jax import lax
from jax.experimental import pallas as pl
from jax.experimental.pallas import tpu as pltpu
from jax.experimental.pallas import tpu_sc as plsc

_NC = 2
_NS = 16
_NW = _NC * _NS
_CHUNK = 64
_LANES = 16


def _pre_body(h_ref, posp_ref, W1_ref, b1_ref, W2p_ref, b2p_ref, Wf3p_ref,
              WfH_ref, bf_ref, U_ref, V_ref):
    h = h_ref[...]
    posp = posp_ref[...]
    t = jnp.maximum(
        jnp.dot(h, W1_ref[...], preferred_element_type=jnp.float32) + b1_ref[...], 0.0)
    deltap = jnp.tanh(
        jnp.dot(t, W2p_ref[...], preferred_element_type=jnp.float32) + b2p_ref[...])
    U_ref[...] = (jnp.dot(h, WfH_ref[...], preferred_element_type=jnp.float32)
                  + jnp.dot(posp, Wf3p_ref[...], preferred_element_type=jnp.float32))
    V_ref[...] = (jnp.dot(deltap - posp, Wf3p_ref[...], preferred_element_type=jnp.float32)
                  + bf_ref[...])


def _make_post_body(n, c, blk):
    def _post_body(p0_ref, p1_ref, h_ref, Wg1_ref, bg1_ref, Wg2_ref,
                   bg2_ref, style_ref, Ws_ref, bs_ref, o_ref,
                   h2_vmem, sum_ref, sumsq_ref):
        ph = pl.program_id(0)
        i = pl.program_id(1)
        row0 = pl.multiple_of(i * blk, 8)

        @pl.when(ph == 0)
        def _():
            aggr = p0_ref[...] + p1_ref[...]
            t = jnp.maximum(
                jnp.dot(aggr, Wg1_ref[...],
                        preferred_element_type=jnp.float32) + bg1_ref[...], 0.0)
            out = jnp.dot(t, Wg2_ref[...],
                          preferred_element_type=jnp.float32) + bg2_ref[...]
            h2 = h_ref[...] + out
            h2 = jnp.where(h2 >= 0, h2, 0.2 * h2)
            h2_vmem[pl.ds(row0, blk), :] = h2
            ps = jnp.sum(h2, axis=0, keepdims=True)
            pss = jnp.sum(h2 * h2, axis=0, keepdims=True)

            @pl.when(i == 0)
            def _():
                sum_ref[...] = ps
                sumsq_ref[...] = pss

            @pl.when(i != 0)
            def _():
                sum_ref[...] += ps
                sumsq_ref[...] += pss

        @pl.when(ph == 1)
        def _():
            mean = sum_ref[...] / n
            var = sumsq_ref[...] / n - mean * mean
            rstd = lax.rsqrt(var + 1e-5)
            s = jnp.dot(style_ref[...], Ws_ref[...],
                        preferred_element_type=jnp.float32) + bs_ref[...]
            gamma = s[:, :c]
            beta = s[:, c:]
            h2 = h2_vmem[pl.ds(row0, blk), :]
            o_ref[...] = gamma * ((h2 - mean) * rstd) + beta
    return _post_body


def _chunk_sizes(total):
    sizes = [_CHUNK] * (total // _CHUNK)
    if total % _CHUNK:
        sizes.append(total % _CHUNK)
    return sizes


def _make_edge_kernel(n, c, e):
    rows_main = ((n // 8) // _NS) * 8
    rows_last = n - (_NS - 1) * rows_main
    nchunks = e // _CHUNK
    iters = (nchunks + _NW - 1) // _NW
    nvec = c // _LANES
    mesh = plsc.VectorSubcoreMesh(core_axis_name="c", subcore_axis_name="s")

    @functools.partial(
        pl.kernel,
        out_type=jax.ShapeDtypeStruct((2 * n, c), jnp.float32),
        mesh=mesh,
        scratch_types=[
            pltpu.VMEM((4, _CHUNK), jnp.int32),
            pltpu.VMEM((4, _CHUNK), jnp.int32),
            pltpu.VMEM((2, _CHUNK, c), jnp.float32),
            pltpu.VMEM((2, _CHUNK, c), jnp.float32),
            pltpu.VMEM((2, _CHUNK, c), jnp.float32),
            pltpu.VMEM_SHARED((n, c), jnp.float32),
            pltpu.SemaphoreType.DMA((4,)),
            pltpu.SemaphoreType.DMA((2,)),
            pltpu.SemaphoreType.DMA((2,)),
        ],
    )
    def _edge(src_hbm, dst_hbm, u_hbm, v_hbm, out_hbm,
              sidx, didx, u_v, v_v, m_v, aggr_sh, semi, semg, semsc):
        ci = lax.axis_index("c")
        si = lax.axis_index("s")
        w = si * _NC + ci

        def cid_of(t):
            return w + t * _NW

        def issue_idx(t, ts):
            q = ts % 4

            @pl.when(cid_of(t) < nchunks)
            def _():
                off = cid_of(t) * _CHUNK
                pltpu.async_copy(src_hbm.at[pl.ds(off, _CHUNK)], sidx.at[q],
                                 semi.at[q])
                pltpu.async_copy(dst_hbm.at[pl.ds(off, _CHUNK)], didx.at[q],
                                 semi.at[q])

        def issue_gathers(t, ts):
            q, b = ts % 4, ts % 2

            @pl.when(cid_of(t) < nchunks)
            def _():
                off = cid_of(t) * _CHUNK
                pltpu.make_async_copy(src_hbm.at[pl.ds(off, _CHUNK)],
                                      sidx.at[q], semi.at[q]).wait()
                pltpu.make_async_copy(dst_hbm.at[pl.ds(off, _CHUNK)],
                                      didx.at[q], semi.at[q]).wait()
                pltpu.async_copy(u_hbm.at[sidx.at[q]], u_v.at[b], semg.at[b])
                pltpu.async_copy(v_hbm.at[didx.at[q]], v_v.at[b], semg.at[b])

        def wait_gathers(t, ts):
            q, b = ts % 4, ts % 2

            @pl.when(cid_of(t) < nchunks)
            def _():
                pltpu.make_async_copy(u_hbm.at[sidx.at[q]], u_v.at[b],
                                      semg.at[b]).wait()
                pltpu.make_async_copy(v_hbm.at[didx.at[q]], v_v.at[b],
                                      semg.at[b]).wait()

        def wait_scatter(ts):
            q, b = ts % 4, ts % 2
            pltpu.make_async_copy(m_v.at[b], aggr_sh.at[didx.at[q]],
                                  semsc.at[b]).wait()

        def drain_scatter(t, ts):
            @pl.when(cid_of(t) < nchunks)
            def _():
                wait_scatter(ts)

        def process(t, ts):
            q, b = ts % 4, ts % 2

            @pl.when(cid_of(t) < nchunks)
            def _():
                @plsc.parallel_loop(0, _CHUNK, unroll=2)
                def _crow(r):
                    for j in range(nvec):
                        sl = pl.ds(j * _LANES, _LANES)
                        m_v[b, r, sl] = jnp.maximum(
                            u_v[b, r, sl] + v_v[b, r, sl], 0.0)
                pltpu.async_copy(m_v.at[b], aggr_sh.at[didx.at[q]],
                                 semsc.at[b], add=True)

        def zrow(r, carry):
            for j in range(nvec):
                m_v[0, r, pl.ds(j * _LANES, _LANES)] = jnp.zeros(
                    (_LANES,), jnp.float32)
            return carry
        lax.fori_loop(0, _CHUNK, zrow, 0)

        base = pl.multiple_of(si * rows_main, 8)

        def _stripe_copy(row_fn):
            @pl.when(si < _NS - 1)
            def _():
                off = 0
                for sz in _chunk_sizes(rows_main):
                    row_fn(off, sz)
                    off += sz

            @pl.when(si == _NS - 1)
            def _():
                off = 0
                for sz in _chunk_sizes(rows_last):
                    row_fn(off, sz)
                    off += sz

        _stripe_copy(lambda off, sz: pltpu.sync_copy(
            m_v.at[0, pl.ds(0, sz)], aggr_sh.at[pl.ds(base + off, sz)]))
        plsc.subcore_barrier()

        issue_idx(0, 0)
        issue_idx(1, 1)
        issue_gathers(0, 0)
        for t0 in (0, 1):
            wait_gathers(t0, t0)
            issue_gathers(t0 + 1, t0 + 1)
            process(t0, t0)
            issue_idx(t0 + 2, t0 + 2)

        nquads = (iters + 3) // 4

        def outer(g, carry):
            for ts in (2, 3, 4, 5):
                t = 4 * g + ts
                wait_gathers(t, ts)
                issue_gathers(t + 1, ts + 1)
                drain_scatter(t - 2, ts - 2)
                process(t, ts)
                issue_idx(t + 2, ts + 2)
            return carry
        lax.fori_loop(0, nquads, outer, 0)

        tlast = 2 + 4 * nquads
        drain_scatter(tlast - 2, tlast - 2)
        drain_scatter(tlast - 1, tlast - 1)
        plsc.subcore_barrier()

        obase = pl.multiple_of(ci * n + base, 8)
        _stripe_copy(lambda off, sz: pltpu.sync_copy(
            aggr_sh.at[pl.ds(base + off, sz)],
            out_hbm.at[pl.ds(obase + off, sz)]))

    return _edge


def kernel(h, pos, edge_index, style, W1, b1, W2, b2, Wf, bf, Wg1, bg1,
           Wg2, bg2, Ws, bs):
    n, c = h.shape
    e = edge_index.shape[1]
    csty = style.shape[1]
    blk = 2000
    grid = n // blk

    src = edge_index[0]
    dst = edge_index[1]
    posp = jnp.pad(pos, ((0, 0), (0, c - pos.shape[1])))
    W2p = jnp.zeros((c, c), jnp.float32).at[:, :pos.shape[1]].set(W2)
    b2p = jnp.zeros((1, c), jnp.float32).at[0, :pos.shape[1]].set(b2)
    Wf3p = jnp.zeros((c, c), jnp.float32).at[:pos.shape[1], :].set(Wf[:pos.shape[1]])
    WfH = Wf[pos.shape[1]:]

    row_spec = pl.BlockSpec((blk, c), lambda i: (i, 0))
    mat_spec = pl.BlockSpec((c, c), lambda i: (0, 0))
    vec_spec = pl.BlockSpec((1, c), lambda i: (0, 0))

    U, V = pl.pallas_call(
        _pre_body,
        grid=(grid,),
        in_specs=[row_spec, row_spec, mat_spec, vec_spec, mat_spec, vec_spec,
                  mat_spec, mat_spec, vec_spec],
        out_specs=[row_spec, row_spec],
        out_shape=[jax.ShapeDtypeStruct((n, c), jnp.float32)] * 2,
    )(h, posp, W1, b1.reshape(1, c), W2p, b2p, Wf3p, WfH, bf.reshape(1, c))

    parts = _make_edge_kernel(n, c, e)(src, dst, U, V)
    p0 = parts[:n]
    p1 = parts[n:]

    def ph0_row(ph, i):
        return (jnp.where(ph == 0, i, grid - 1), 0)

    def ph1_row(ph, i):
        return (jnp.where(ph == 0, 0, i), 0)

    const2 = lambda ph, i: (0, 0)
    out = pl.pallas_call(
        _make_post_body(n, c, blk),
        grid=(2, grid),
        in_specs=[pl.BlockSpec((blk, c), ph0_row),
                  pl.BlockSpec((blk, c), ph0_row),
                  pl.BlockSpec((blk, c), ph0_row),
                  pl.BlockSpec((c, c), const2),
                  pl.BlockSpec((1, c), const2),
                  pl.BlockSpec((c, c), const2),
                  pl.BlockSpec((1, c), const2),
                  pl.BlockSpec((blk, csty), ph1_row),
                  pl.BlockSpec((csty, 2 * c), const2),
                  pl.BlockSpec((1, 2 * c), const2)],
        out_specs=pl.BlockSpec((blk, c), ph1_row),
        out_shape=jax.ShapeDtypeStruct((n, c), jnp.float32),
        scratch_shapes=[pltpu.VMEM((n, c), jnp.float32),
                        pltpu.VMEM((1, c), jnp.float32),
                        pltpu.VMEM((1, c), jnp.float32)],
    )(p0, p1, h, Wg1, bg1.reshape(1, c), Wg2, bg2.reshape(1, c),
      style, Ws, bs.reshape(1, 2 * c))
    return out


# speedup vs baseline: 13.6118x; 1.0223x over previous
import functools

import jax
import jax.numpy as jnp
from jax import lax
from jax.experimental import pallas as pl
from jax.experimental.pallas import tpu as pltpu
from jax.experimental.pallas import tpu_sc as plsc

_NC = 2
_NS = 16
_NW = _NC * _NS
_CHUNK = 64
_LANES = 16


def _pre_body(h_ref, posp_ref, W1_ref, b1_ref, W2p_ref, b2p_ref, Wf3p_ref,
              WfH_ref, bf_ref, U_ref, V_ref):
    h = h_ref[...]
    posp = posp_ref[...]
    t = jnp.maximum(
        jnp.dot(h, W1_ref[...], preferred_element_type=jnp.float32) + b1_ref[...], 0.0)
    deltap = jnp.tanh(
        jnp.dot(t, W2p_ref[...], preferred_element_type=jnp.float32) + b2p_ref[...])
    u = (jnp.dot(h, WfH_ref[...], preferred_element_type=jnp.float32)
         + jnp.dot(posp, Wf3p_ref[...], preferred_element_type=jnp.float32))
    v = (jnp.dot(deltap - posp, Wf3p_ref[...], preferred_element_type=jnp.float32)
         + bf_ref[...])
    U_ref[...] = u
    V_ref[...] = v


def _make_post_body(n, c, blk):
    def _post_body(p0_ref, p1_ref, h_ref, Wg1_ref, bg1_ref,
                   Wg2_ref, bg2_ref, style_ref, Ws_ref, bs_ref, o_ref,
                   h2_vmem, sum_ref, sumsq_ref):
        ph = pl.program_id(0)
        i = pl.program_id(1)
        row0 = pl.multiple_of(i * blk, 8)

        @pl.when(ph == 0)
        def _():
            aggr = p0_ref[...] + p1_ref[...]
            t = jnp.maximum(
                jnp.dot(aggr, Wg1_ref[...],
                        preferred_element_type=jnp.float32) + bg1_ref[...], 0.0)
            out = jnp.dot(t, Wg2_ref[...],
                          preferred_element_type=jnp.float32) + bg2_ref[...]
            h2 = h_ref[...] + out
            h2 = jnp.where(h2 >= 0, h2, 0.2 * h2)
            h2_vmem[pl.ds(row0, blk), :] = h2
            ps = jnp.sum(h2, axis=0, keepdims=True)
            pss = jnp.sum(h2 * h2, axis=0, keepdims=True)

            @pl.when(i == 0)
            def _():
                sum_ref[...] = ps
                sumsq_ref[...] = pss

            @pl.when(i != 0)
            def _():
                sum_ref[...] += ps
                sumsq_ref[...] += pss

        @pl.when(ph == 1)
        def _():
            mean = sum_ref[...] / n
            var = sumsq_ref[...] / n - mean * mean
            rstd = lax.rsqrt(var + 1e-5)
            s = jnp.dot(style_ref[...], Ws_ref[...],
                        preferred_element_type=jnp.float32) + bs_ref[...]
            gamma = s[:, :c]
            beta = s[:, c:]
            h2 = h2_vmem[pl.ds(row0, blk), :]
            o_ref[...] = gamma * ((h2 - mean) * rstd) + beta
    return _post_body


def _chunk_sizes(total):
    sizes = [_CHUNK] * (total // _CHUNK)
    if total % _CHUNK:
        sizes.append(total % _CHUNK)
    return sizes


def _make_edge_kernel(n, c, e):
    rows_main = ((n // 8) // _NS) * 8
    rows_last = n - (_NS - 1) * rows_main
    nchunks = e // _CHUNK
    nworkers = _NW
    iters = (nchunks + nworkers - 1) // nworkers
    nvec = c // _LANES
    mesh = plsc.VectorSubcoreMesh(core_axis_name="c", subcore_axis_name="s")

    @functools.partial(
        pl.kernel,
        out_type=[jax.ShapeDtypeStruct((n, c), jnp.float32),
                  jax.ShapeDtypeStruct((n, c), jnp.float32)],
        mesh=mesh,
        scratch_types=[
            pltpu.VMEM((4, _CHUNK), jnp.int32),
            pltpu.VMEM((4, _CHUNK), jnp.int32),
            pltpu.VMEM((2, _CHUNK, c), jnp.float32),
            pltpu.VMEM((2, _CHUNK, c), jnp.float32),
            pltpu.VMEM((2, _CHUNK, c), jnp.float32),
            pltpu.VMEM_SHARED((n, c), jnp.float32),
            pltpu.SemaphoreType.DMA((4,)),
            pltpu.SemaphoreType.DMA((2,)),
            pltpu.SemaphoreType.DMA((2,)),
        ],
    )
    def _edge(src_hbm, dst_hbm, u_hbm, v_hbm, out0_hbm, out1_hbm,
              sidx, didx, u_v, v_v, m_v, aggr_sh, semi, semg, semsc):
        ci = lax.axis_index("c")
        si = lax.axis_index("s")
        w = si * _NC + ci

        def cid_of(t):
            return w + t * nworkers

        def issue_idx(t, ts):
            q = ts % 4

            @pl.when(cid_of(t) < nchunks)
            def _():
                off = cid_of(t) * _CHUNK
                pltpu.async_copy(src_hbm.at[pl.ds(off, _CHUNK)], sidx.at[q],
                                 semi.at[q])
                pltpu.async_copy(dst_hbm.at[pl.ds(off, _CHUNK)], didx.at[q],
                                 semi.at[q])

        def issue_gathers(t, ts):
            q, b = ts % 4, ts % 2

            @pl.when(cid_of(t) < nchunks)
            def _():
                off = cid_of(t) * _CHUNK
                pltpu.make_async_copy(src_hbm.at[pl.ds(off, _CHUNK)],
                                      sidx.at[q], semi.at[q]).wait()
                pltpu.make_async_copy(dst_hbm.at[pl.ds(off, _CHUNK)],
                                      didx.at[q], semi.at[q]).wait()
                pltpu.async_copy(u_hbm.at[sidx.at[q]], u_v.at[b], semg.at[b])
                pltpu.async_copy(v_hbm.at[didx.at[q]], v_v.at[b], semg.at[b])

        def wait_gathers(t, ts):
            q, b = ts % 4, ts % 2

            @pl.when(cid_of(t) < nchunks)
            def _():
                pltpu.make_async_copy(u_hbm.at[sidx.at[q]], u_v.at[b],
                                      semg.at[b]).wait()
                pltpu.make_async_copy(v_hbm.at[didx.at[q]], v_v.at[b],
                                      semg.at[b]).wait()

        def wait_scatter(ts):
            q, b = ts % 4, ts % 2
            pltpu.make_async_copy(m_v.at[b], aggr_sh.at[didx.at[q]],
                                  semsc.at[b]).wait()

        def drain_scatter(t, ts):
            @pl.when(cid_of(t) < nchunks)
            def _():
                wait_scatter(ts)

        def process(t, ts):
            q, b = ts % 4, ts % 2

            @pl.when(cid_of(t) < nchunks)
            def _():
                @plsc.parallel_loop(0, _CHUNK, unroll=2)
                def _crow(r):
                    for j in range(nvec):
                        sl = pl.ds(j * _LANES, _LANES)
                        m_v[b, r, sl] = jnp.maximum(
                            u_v[b, r, sl] + v_v[b, r, sl], 0.0)
                pltpu.async_copy(m_v.at[b], aggr_sh.at[didx.at[q]],
                                 semsc.at[b], add=True)

        def zrow(r, carry):
            for j in range(nvec):
                m_v[0, r, pl.ds(j * _LANES, _LANES)] = jnp.zeros(
                    (_LANES,), jnp.float32)
            return carry
        lax.fori_loop(0, _CHUNK, zrow, 0)

        base = pl.multiple_of(si * rows_main, 8)

        def _stripe_copy(row_fn):
            @pl.when(si < _NS - 1)
            def _():
                off = 0
                for sz in _chunk_sizes(rows_main):
                    row_fn(off, sz)
                    off += sz

            @pl.when(si == _NS - 1)
            def _():
                off = 0
                for sz in _chunk_sizes(rows_last):
                    row_fn(off, sz)
                    off += sz

        _stripe_copy(lambda off, sz: pltpu.sync_copy(
            m_v.at[0, pl.ds(0, sz)], aggr_sh.at[pl.ds(base + off, sz)]))
        plsc.subcore_barrier()

        issue_idx(0, 0)
        issue_idx(1, 1)
        issue_gathers(0, 0)
        for t0 in (0, 1):
            wait_gathers(t0, t0)
            issue_gathers(t0 + 1, t0 + 1)
            process(t0, t0)
            issue_idx(t0 + 2, t0 + 2)

        nquads = (iters + 3) // 4

        def outer(g, carry):
            for ts in (2, 3, 4, 5):
                t = 4 * g + ts
                wait_gathers(t, ts)
                issue_gathers(t + 1, ts + 1)
                drain_scatter(t - 2, ts - 2)
                process(t, ts)
                issue_idx(t + 2, ts + 2)
            return carry
        lax.fori_loop(0, nquads, outer, 0)

        tlast = 2 + 4 * nquads
        drain_scatter(tlast - 2, tlast - 2)
        drain_scatter(tlast - 1, tlast - 1)
        plsc.subcore_barrier()

        @pl.when(ci == 0)
        def _():
            _stripe_copy(lambda off, sz: pltpu.sync_copy(
                aggr_sh.at[pl.ds(base + off, sz)],
                out0_hbm.at[pl.ds(base + off, sz)]))

        @pl.when(ci == 1)
        def _():
            _stripe_copy(lambda off, sz: pltpu.sync_copy(
                aggr_sh.at[pl.ds(base + off, sz)],
                out1_hbm.at[pl.ds(base + off, sz)]))

    return _edge


def kernel(h, pos, edge_index, style, W1, b1, W2, b2, Wf, bf, Wg1, bg1,
           Wg2, bg2, Ws, bs):
    n, c = h.shape
    e = edge_index.shape[1]
    csty = style.shape[1]
    blk = 2000
    grid = n // blk

    src = edge_index[0]
    dst = edge_index[1]
    posp = jnp.pad(pos, ((0, 0), (0, c - pos.shape[1])))
    W2p = jnp.zeros((c, c), jnp.float32).at[:, :pos.shape[1]].set(W2)
    b2p = jnp.zeros((1, c), jnp.float32).at[0, :pos.shape[1]].set(b2)
    Wf3p = jnp.zeros((c, c), jnp.float32).at[:pos.shape[1], :].set(Wf[:pos.shape[1]])
    WfH = Wf[pos.shape[1]:]

    row_spec = pl.BlockSpec((blk, c), lambda i: (i, 0))
    mat_spec = pl.BlockSpec((c, c), lambda i: (0, 0))
    vec_spec = pl.BlockSpec((1, c), lambda i: (0, 0))

    U, V = pl.pallas_call(
        _pre_body,
        grid=(grid,),
        in_specs=[row_spec, row_spec, mat_spec, vec_spec, mat_spec, vec_spec,
                  mat_spec, mat_spec, vec_spec],
        out_specs=[row_spec, row_spec],
        out_shape=[jax.ShapeDtypeStruct((n, c), jnp.float32)] * 2,
    )(h, posp, W1, b1.reshape(1, c), W2p, b2p, Wf3p, WfH, bf.reshape(1, c))

    p0, p1 = _make_edge_kernel(n, c, e)(src, dst, U, V)

    def ph0_row(ph, i):
        return (jnp.where(ph == 0, i, grid - 1), 0)

    def ph1_row(ph, i):
        return (jnp.where(ph == 0, 0, i), 0)

    const2 = lambda ph, i: (0, 0)
    out = pl.pallas_call(
        _make_post_body(n, c, blk),
        grid=(2, grid),
        in_specs=[pl.BlockSpec((blk, c), ph0_row),
                  pl.BlockSpec((blk, c), ph0_row),
                  pl.BlockSpec((blk, c), ph0_row),
                  pl.BlockSpec((c, c), const2),
                  pl.BlockSpec((1, c), const2),
                  pl.BlockSpec((c, c), const2),
                  pl.BlockSpec((1, c), const2),
                  pl.BlockSpec((blk, csty), ph1_row),
                  pl.BlockSpec((csty, 2 * c), const2),
                  pl.BlockSpec((1, 2 * c), const2)],
        out_specs=pl.BlockSpec((blk, c), ph1_row),
        out_shape=jax.ShapeDtypeStruct((n, c), jnp.float32),
        scratch_shapes=[pltpu.VMEM((n, c), jnp.float32),
                        pltpu.VMEM((1, c), jnp.float32),
                        pltpu.VMEM((1, c), jnp.float32)],
    )(p0, p1, h, Wg1, bg1.reshape(1, c), Wg2,
      bg2.reshape(1, c), style, Ws, bs.reshape(1, 2 * c))
    return out


# speedup vs baseline: 13.7606x; 1.0109x over previous
import functools

import jax
import jax.numpy as jnp
from jax import lax
from jax.experimental import pallas as pl
from jax.experimental.pallas import tpu as pltpu
from jax.experimental.pallas import tpu_sc as plsc

_NC = 2
_NS = 16
_NW = _NC * _NS
_CHUNK = 64
_LANES = 16


def _pre_body(h_ref, posp_ref, W1_ref, b1_ref, W2p_ref, b2p_ref, Wf3p_ref,
              WfH_ref, bf_ref, U_ref, V_ref):
    h = h_ref[...]
    posp = posp_ref[...]
    t = jnp.maximum(
        jnp.dot(h, W1_ref[...], preferred_element_type=jnp.float32) + b1_ref[...], 0.0)
    deltap = jnp.tanh(
        jnp.dot(t, W2p_ref[...], preferred_element_type=jnp.float32) + b2p_ref[...])
    u = (jnp.dot(h, WfH_ref[...], preferred_element_type=jnp.float32)
         + jnp.dot(posp, Wf3p_ref[...], preferred_element_type=jnp.float32))
    v = (jnp.dot(deltap - posp, Wf3p_ref[...], preferred_element_type=jnp.float32)
         + bf_ref[...])
    U_ref[...] = u
    V_ref[...] = v


def _make_post_body(n, c, blk):
    def _post_body(p0_ref, p1_ref, h_ref, Wg1_ref, bg1_ref,
                   Wg2_ref, bg2_ref, style_ref, Ws_ref, bs_ref, o_ref,
                   h2_vmem, sum_ref, sumsq_ref):
        ph = pl.program_id(0)
        i = pl.program_id(1)
        row0 = pl.multiple_of(i * blk, 8)

        @pl.when(ph == 0)
        def _():
            aggr = p0_ref[...] + p1_ref[...]
            t = jnp.maximum(
                jnp.dot(aggr, Wg1_ref[...],
                        preferred_element_type=jnp.float32) + bg1_ref[...], 0.0)
            out = jnp.dot(t, Wg2_ref[...],
                          preferred_element_type=jnp.float32) + bg2_ref[...]
            h2 = h_ref[...] + out
            h2 = jnp.where(h2 >= 0, h2, 0.2 * h2)
            h2_vmem[pl.ds(row0, blk), :] = h2
            ps = jnp.sum(h2, axis=0, keepdims=True)
            pss = jnp.sum(h2 * h2, axis=0, keepdims=True)

            @pl.when(i == 0)
            def _():
                sum_ref[...] = ps
                sumsq_ref[...] = pss

            @pl.when(i != 0)
            def _():
                sum_ref[...] += ps
                sumsq_ref[...] += pss

        @pl.when(ph == 1)
        def _():
            mean = sum_ref[...] / n
            var = sumsq_ref[...] / n - mean * mean
            rstd = lax.rsqrt(var + 1e-5)
            s = jnp.dot(style_ref[...], Ws_ref[...],
                        preferred_element_type=jnp.float32) + bs_ref[...]
            gamma = s[:, :c]
            beta = s[:, c:]
            h2 = h2_vmem[pl.ds(row0, blk), :]
            o_ref[...] = gamma * ((h2 - mean) * rstd) + beta
    return _post_body


def _chunk_sizes(total):
    sizes = [_CHUNK] * (total // _CHUNK)
    if total % _CHUNK:
        sizes.append(total % _CHUNK)
    return sizes


def _make_edge_kernel(n, c, e):
    rows_main = ((n // 8) // _NS) * 8
    rows_last = n - (_NS - 1) * rows_main
    nchunks = e // _CHUNK
    nworkers = _NW
    iters = (nchunks + nworkers - 1) // nworkers
    nvec = c // _LANES
    mesh = plsc.VectorSubcoreMesh(core_axis_name="c", subcore_axis_name="s")

    @functools.partial(
        pl.kernel,
        out_type=[jax.ShapeDtypeStruct((n, c), jnp.float32),
                  jax.ShapeDtypeStruct((n, c), jnp.float32)],
        mesh=mesh,
        scratch_types=[
            pltpu.VMEM((4, _CHUNK), jnp.int32),
            pltpu.VMEM((4, _CHUNK), jnp.int32),
            pltpu.VMEM((2, _CHUNK, c), jnp.float32),
            pltpu.VMEM((2, _CHUNK, c), jnp.float32),
            pltpu.VMEM((2, _CHUNK, c), jnp.float32),
            pltpu.VMEM_SHARED((n, c), jnp.float32),
            pltpu.SemaphoreType.DMA((4,)),
            pltpu.SemaphoreType.DMA((2,)),
            pltpu.SemaphoreType.DMA((2,)),
        ],
    )
    def _edge(src_hbm, dst_hbm, u_hbm, v_hbm, out0_hbm, out1_hbm,
              sidx, didx, u_v, v_v, m_v, aggr_sh, semi, semg, semsc):
        ci = lax.axis_index("c")
        si = lax.axis_index("s")
        w = si * _NC + ci

        def cid_of(t):
            return w + t * nworkers

        def issue_idx(t, ts):
            q = ts % 4

            @pl.when(cid_of(t) < nchunks)
            def _():
                off = cid_of(t) * _CHUNK
                pltpu.async_copy(src_hbm.at[pl.ds(off, _CHUNK)], sidx.at[q],
                                 semi.at[q])
                pltpu.async_copy(dst_hbm.at[pl.ds(off, _CHUNK)], didx.at[q],
                                 semi.at[q])

        def issue_gathers(t, ts):
            q, b = ts % 4, ts % 2

            @pl.when(cid_of(t) < nchunks)
            def _():
                off = cid_of(t) * _CHUNK
                pltpu.make_async_copy(src_hbm.at[pl.ds(off, _CHUNK)],
                                      sidx.at[q], semi.at[q]).wait()
                pltpu.make_async_copy(dst_hbm.at[pl.ds(off, _CHUNK)],
                                      didx.at[q], semi.at[q]).wait()
                pltpu.async_copy(u_hbm.at[sidx.at[q]], u_v.at[b], semg.at[b])
                pltpu.async_copy(v_hbm.at[didx.at[q]], v_v.at[b], semg.at[b])

        def wait_gathers(t, ts):
            q, b = ts % 4, ts % 2

            @pl.when(cid_of(t) < nchunks)
            def _():
                pltpu.make_async_copy(u_hbm.at[sidx.at[q]], u_v.at[b],
                                      semg.at[b]).wait()
                pltpu.make_async_copy(v_hbm.at[didx.at[q]], v_v.at[b],
                                      semg.at[b]).wait()

        def wait_scatter(ts):
            q, b = ts % 4, ts % 2
            pltpu.make_async_copy(m_v.at[b], aggr_sh.at[didx.at[q]],
                                  semsc.at[b]).wait()

        def drain_scatter(t, ts):
            @pl.when(cid_of(t) < nchunks)
            def _():
                wait_scatter(ts)

        def process(t, ts):
            q, b = ts % 4, ts % 2

            @pl.when(cid_of(t) < nchunks)
            def _():
                @plsc.parallel_loop(0, _CHUNK, unroll=2)
                def _crow(r):
                    for j in range(nvec):
                        sl = pl.ds(j * _LANES, _LANES)
                        m_v[b, r, sl] = jnp.maximum(
                            u_v[b, r, sl] + v_v[b, r, sl], 0.0)
                pltpu.async_copy(m_v.at[b], aggr_sh.at[didx.at[q]],
                                 semsc.at[b], add=True)

        def zrow(r, carry):
            for j in range(nvec):
                m_v[0, r, pl.ds(j * _LANES, _LANES)] = jnp.zeros(
                    (_LANES,), jnp.float32)
            return carry
        lax.fori_loop(0, _CHUNK, zrow, 0)

        base = pl.multiple_of(si * rows_main, 8)

        def _stripe_copy(row_fn):
            @pl.when(si < _NS - 1)
            def _():
                off = 0
                for sz in _chunk_sizes(rows_main):
                    row_fn(off, sz)
                    off += sz

            @pl.when(si == _NS - 1)
            def _():
                off = 0
                for sz in _chunk_sizes(rows_last):
                    row_fn(off, sz)
                    off += sz

        _stripe_copy(lambda off, sz: pltpu.async_copy(
            m_v.at[0, pl.ds(0, sz)], aggr_sh.at[pl.ds(base + off, sz)],
            semi.at[0]))
        _stripe_copy(lambda off, sz: pltpu.make_async_copy(
            m_v.at[0, pl.ds(0, sz)], aggr_sh.at[pl.ds(base + off, sz)],
            semi.at[0]).wait())
        plsc.subcore_barrier()

        issue_idx(0, 0)
        issue_idx(1, 1)
        issue_gathers(0, 0)
        for t0 in (0, 1):
            wait_gathers(t0, t0)
            issue_gathers(t0 + 1, t0 + 1)
            process(t0, t0)
            issue_idx(t0 + 2, t0 + 2)

        nquads = (iters + 3) // 4

        def outer(g, carry):
            for ts in (2, 3, 4, 5):
                t = 4 * g + ts
                wait_gathers(t, ts)
                issue_gathers(t + 1, ts + 1)
                drain_scatter(t - 2, ts - 2)
                process(t, ts)
                issue_idx(t + 2, ts + 2)
            return carry
        lax.fori_loop(0, nquads, outer, 0)

        tlast = 2 + 4 * nquads
        drain_scatter(tlast - 2, tlast - 2)
        drain_scatter(tlast - 1, tlast - 1)
        plsc.subcore_barrier()

        @pl.when(ci == 0)
        def _():
            _stripe_copy(lambda off, sz: pltpu.async_copy(
                aggr_sh.at[pl.ds(base + off, sz)],
                out0_hbm.at[pl.ds(base + off, sz)], semi.at[0]))
            _stripe_copy(lambda off, sz: pltpu.make_async_copy(
                aggr_sh.at[pl.ds(base + off, sz)],
                out0_hbm.at[pl.ds(base + off, sz)], semi.at[0]).wait())

        @pl.when(ci == 1)
        def _():
            _stripe_copy(lambda off, sz: pltpu.async_copy(
                aggr_sh.at[pl.ds(base + off, sz)],
                out1_hbm.at[pl.ds(base + off, sz)], semi.at[0]))
            _stripe_copy(lambda off, sz: pltpu.make_async_copy(
                aggr_sh.at[pl.ds(base + off, sz)],
                out1_hbm.at[pl.ds(base + off, sz)], semi.at[0]).wait())

    return _edge


def kernel(h, pos, edge_index, style, W1, b1, W2, b2, Wf, bf, Wg1, bg1,
           Wg2, bg2, Ws, bs):
    n, c = h.shape
    e = edge_index.shape[1]
    csty = style.shape[1]
    blk = 5000
    grid = n // blk

    src = edge_index[0]
    dst = edge_index[1]
    posp = jnp.pad(pos, ((0, 0), (0, c - pos.shape[1])))
    W2p = jnp.zeros((c, c), jnp.float32).at[:, :pos.shape[1]].set(W2)
    b2p = jnp.zeros((1, c), jnp.float32).at[0, :pos.shape[1]].set(b2)
    Wf3p = jnp.zeros((c, c), jnp.float32).at[:pos.shape[1], :].set(Wf[:pos.shape[1]])
    WfH = Wf[pos.shape[1]:]

    row_spec = pl.BlockSpec((blk, c), lambda i: (i, 0))
    mat_spec = pl.BlockSpec((c, c), lambda i: (0, 0))
    vec_spec = pl.BlockSpec((1, c), lambda i: (0, 0))

    U, V = pl.pallas_call(
        _pre_body,
        grid=(grid,),
        in_specs=[row_spec, row_spec, mat_spec, vec_spec, mat_spec, vec_spec,
                  mat_spec, mat_spec, vec_spec],
        out_specs=[row_spec, row_spec],
        out_shape=[jax.ShapeDtypeStruct((n, c), jnp.float32)] * 2,
    )(h, posp, W1, b1.reshape(1, c), W2p, b2p, Wf3p, WfH, bf.reshape(1, c))

    p0, p1 = _make_edge_kernel(n, c, e)(src, dst, U, V)

    def ph0_row(ph, i):
        return (jnp.where(ph == 0, i, grid - 1), 0)

    def ph1_row(ph, i):
        return (jnp.where(ph == 0, 0, i), 0)

    const2 = lambda ph, i: (0, 0)
    out = pl.pallas_call(
        _make_post_body(n, c, blk),
        grid=(2, grid),
        in_specs=[pl.BlockSpec((blk, c), ph0_row),
                  pl.BlockSpec((blk, c), ph0_row),
                  pl.BlockSpec((blk, c), ph0_row),
                  pl.BlockSpec((c, c), const2),
                  pl.BlockSpec((1, c), const2),
                  pl.BlockSpec((c, c), const2),
                  pl.BlockSpec((1, c), const2),
                  pl.BlockSpec((blk, csty), ph1_row),
                  pl.BlockSpec((csty, 2 * c), const2),
                  pl.BlockSpec((1, 2 * c), const2)],
        out_specs=pl.BlockSpec((blk, c), ph1_row),
        out_shape=jax.ShapeDtypeStruct((n, c), jnp.float32),
        scratch_shapes=[pltpu.VMEM((n, c), jnp.float32),
                        pltpu.VMEM((1, c), jnp.float32),
                        pltpu.VMEM((1, c), jnp.float32)],
    )(p0, p1, h, Wg1, bg1.reshape(1, c), Wg2,
      bg2.reshape(1, c), style, Ws, bs.reshape(1, 2 * c))
    return out


# speedup vs baseline: 16.8714x; 1.2261x over previous
import functools

import jax
import jax.numpy as jnp
from jax import lax
from jax.experimental import pallas as pl
from jax.experimental.pallas import tpu as pltpu
from jax.experimental.pallas import tpu_sc as plsc

_NC = 2
_NS = 16
_NW = _NC * _NS
_CHUNK = 40
_LANES = 16


def _pre_body(h_ref, posp_ref, W1_ref, b1_ref, W2p_ref, b2p_ref, Wf3p_ref,
              WfH_ref, bf_ref, U_ref, V_ref):
    h = h_ref[...]
    posp = posp_ref[...]
    t = jnp.maximum(
        jnp.dot(h, W1_ref[...], preferred_element_type=jnp.float32) + b1_ref[...], 0.0)
    deltap = jnp.tanh(
        jnp.dot(t, W2p_ref[...], preferred_element_type=jnp.float32) + b2p_ref[...])
    u = (jnp.dot(h, WfH_ref[...], preferred_element_type=jnp.float32)
         + jnp.dot(posp, Wf3p_ref[...], preferred_element_type=jnp.float32))
    v = (jnp.dot(deltap - posp, Wf3p_ref[...], preferred_element_type=jnp.float32)
         + bf_ref[...])
    U_ref[...] = u
    V_ref[...] = v


def _make_post_body(n, c, blk):
    def _post_body(p0_ref, p1_ref, h_ref, Wg1_ref, bg1_ref,
                   Wg2_ref, bg2_ref, style_ref, Ws_ref, bs_ref, o_ref,
                   h2_vmem, sum_ref, sumsq_ref):
        ph = pl.program_id(0)
        i = pl.program_id(1)
        row0 = pl.multiple_of(i * blk, 8)

        @pl.when(ph == 0)
        def _():
            aggr = p0_ref[...] + p1_ref[...]
            t = jnp.maximum(
                jnp.dot(aggr, Wg1_ref[...],
                        preferred_element_type=jnp.float32) + bg1_ref[...], 0.0)
            out = jnp.dot(t, Wg2_ref[...],
                          preferred_element_type=jnp.float32) + bg2_ref[...]
            h2 = h_ref[...] + out
            h2 = jnp.where(h2 >= 0, h2, 0.2 * h2)
            h2_vmem[pl.ds(row0, blk), :] = h2
            ps = jnp.sum(h2, axis=0, keepdims=True)
            pss = jnp.sum(h2 * h2, axis=0, keepdims=True)

            @pl.when(i == 0)
            def _():
                sum_ref[...] = ps
                sumsq_ref[...] = pss

            @pl.when(i != 0)
            def _():
                sum_ref[...] += ps
                sumsq_ref[...] += pss

        @pl.when(ph == 1)
        def _():
            mean = sum_ref[...] / n
            var = sumsq_ref[...] / n - mean * mean
            rstd = lax.rsqrt(var + 1e-5)
            s = jnp.dot(style_ref[...], Ws_ref[...],
                        preferred_element_type=jnp.float32) + bs_ref[...]
            gamma = s[:, :c]
            beta = s[:, c:]
            h2 = h2_vmem[pl.ds(row0, blk), :]
            o_ref[...] = gamma * ((h2 - mean) * rstd) + beta
    return _post_body


def _chunk_sizes(total):
    sizes = [_CHUNK] * (total // _CHUNK)
    if total % _CHUNK:
        sizes.append(total % _CHUNK)
    return sizes


def _make_edge_kernel(n, c, e):
    rows_main = ((n // 8) // _NS) * 8
    rows_last = n - (_NS - 1) * rows_main
    nchunks = e // _CHUNK
    nworkers = _NW
    iters = (nchunks + nworkers - 1) // nworkers
    nvec = c // _LANES
    mesh = plsc.VectorSubcoreMesh(core_axis_name="c", subcore_axis_name="s")

    @functools.partial(
        pl.kernel,
        out_type=[jax.ShapeDtypeStruct((n, c), jnp.float32),
                  jax.ShapeDtypeStruct((n, c), jnp.float32)],
        mesh=mesh,
        scratch_types=[
            pltpu.VMEM((6, _CHUNK), jnp.int32),
            pltpu.VMEM((6, _CHUNK), jnp.int32),
            pltpu.VMEM((3, _CHUNK, c), jnp.float32),
            pltpu.VMEM((3, _CHUNK, c), jnp.float32),
            pltpu.VMEM((2, _CHUNK, c), jnp.float32),
            pltpu.VMEM_SHARED((n, c), jnp.float32),
            pltpu.SemaphoreType.DMA((6,)),
            pltpu.SemaphoreType.DMA((3,)),
            pltpu.SemaphoreType.DMA((2,)),
        ],
    )
    def _edge(src_hbm, dst_hbm, u_hbm, v_hbm, out0_hbm, out1_hbm,
              sidx, didx, u_v, v_v, m_v, aggr_sh, semi, semg, semsc):
        ci = lax.axis_index("c")
        si = lax.axis_index("s")
        w = si * _NC + ci

        def cid_of(t):
            return w + t * nworkers

        def issue_idx(t, ts):
            q = ts % 6

            @pl.when(cid_of(t) < nchunks)
            def _():
                off = cid_of(t) * _CHUNK
                pltpu.async_copy(src_hbm.at[pl.ds(off, _CHUNK)], sidx.at[q],
                                 semi.at[q])
                pltpu.async_copy(dst_hbm.at[pl.ds(off, _CHUNK)], didx.at[q],
                                 semi.at[q])

        def issue_gathers(t, ts):
            q, b = ts % 6, ts % 3

            @pl.when(cid_of(t) < nchunks)
            def _():
                off = cid_of(t) * _CHUNK
                pltpu.make_async_copy(src_hbm.at[pl.ds(off, _CHUNK)],
                                      sidx.at[q], semi.at[q]).wait()
                pltpu.make_async_copy(dst_hbm.at[pl.ds(off, _CHUNK)],
                                      didx.at[q], semi.at[q]).wait()
                pltpu.async_copy(u_hbm.at[sidx.at[q]], u_v.at[b], semg.at[b])
                pltpu.async_copy(v_hbm.at[didx.at[q]], v_v.at[b], semg.at[b])

        def wait_gathers(t, ts):
            q, b = ts % 6, ts % 3

            @pl.when(cid_of(t) < nchunks)
            def _():
                pltpu.make_async_copy(u_hbm.at[sidx.at[q]], u_v.at[b],
                                      semg.at[b]).wait()
                pltpu.make_async_copy(v_hbm.at[didx.at[q]], v_v.at[b],
                                      semg.at[b]).wait()

        def wait_scatter(ts):
            q, b = ts % 6, ts % 2
            pltpu.make_async_copy(m_v.at[b], aggr_sh.at[didx.at[q]],
                                  semsc.at[b]).wait()

        def drain_scatter(t, ts):
            @pl.when(cid_of(t) < nchunks)
            def _():
                wait_scatter(ts)

        def process(t, ts):
            q, b3, b2 = ts % 6, ts % 3, ts % 2

            @pl.when(cid_of(t) < nchunks)
            def _():
                @plsc.parallel_loop(0, _CHUNK, unroll=2)
                def _crow(r):
                    for j in range(nvec):
                        sl = pl.ds(j * _LANES, _LANES)
                        m_v[b2, r, sl] = jnp.maximum(
                            u_v[b3, r, sl] + v_v[b3, r, sl], 0.0)
                pltpu.async_copy(m_v.at[b2], aggr_sh.at[didx.at[q]],
                                 semsc.at[b2], add=True)

        def zrow(r, carry):
            for j in range(nvec):
                m_v[0, r, pl.ds(j * _LANES, _LANES)] = jnp.zeros(
                    (_LANES,), jnp.float32)
            return carry
        lax.fori_loop(0, _CHUNK, zrow, 0)

        base = pl.multiple_of(si * rows_main, 8)

        def _stripe_copy(row_fn):
            @pl.when(si < _NS - 1)
            def _():
                off = 0
                for sz in _chunk_sizes(rows_main):
                    row_fn(off, sz)
                    off += sz

            @pl.when(si == _NS - 1)
            def _():
                off = 0
                for sz in _chunk_sizes(rows_last):
                    row_fn(off, sz)
                    off += sz

        _stripe_copy(lambda off, sz: pltpu.async_copy(
            m_v.at[0, pl.ds(0, sz)], aggr_sh.at[pl.ds(base + off, sz)],
            semi.at[0]))
        _stripe_copy(lambda off, sz: pltpu.make_async_copy(
            m_v.at[0, pl.ds(0, sz)], aggr_sh.at[pl.ds(base + off, sz)],
            semi.at[0]).wait())
        plsc.subcore_barrier()

        for t0 in (0, 1, 2, 3):
            issue_idx(t0, t0)
        issue_gathers(0, 0)
        issue_gathers(1, 1)
        for t0 in (0, 1):
            wait_gathers(t0, t0)
            issue_gathers(t0 + 2, t0 + 2)
            process(t0, t0)
            issue_idx(t0 + 4, t0 + 4)

        nhex = (iters + 5) // 6

        def outer(g, carry):
            for ts in (2, 3, 4, 5, 6, 7):
                t = 6 * g + ts
                wait_gathers(t, ts)
                issue_gathers(t + 2, ts + 2)
                drain_scatter(t - 2, ts - 2)
                process(t, ts)
                issue_idx(t + 4, ts + 4)
            return carry
        lax.fori_loop(0, nhex, outer, 0)

        tlast = 2 + 6 * nhex
        drain_scatter(tlast - 2, tlast - 2)
        drain_scatter(tlast - 1, tlast - 1)
        plsc.subcore_barrier()

        @pl.when(ci == 0)
        def _():
            _stripe_copy(lambda off, sz: pltpu.async_copy(
                aggr_sh.at[pl.ds(base + off, sz)],
                out0_hbm.at[pl.ds(base + off, sz)], semi.at[0]))
            _stripe_copy(lambda off, sz: pltpu.make_async_copy(
                aggr_sh.at[pl.ds(base + off, sz)],
                out0_hbm.at[pl.ds(base + off, sz)], semi.at[0]).wait())

        @pl.when(ci == 1)
        def _():
            _stripe_copy(lambda off, sz: pltpu.async_copy(
                aggr_sh.at[pl.ds(base + off, sz)],
                out1_hbm.at[pl.ds(base + off, sz)], semi.at[0]))
            _stripe_copy(lambda off, sz: pltpu.make_async_copy(
                aggr_sh.at[pl.ds(base + off, sz)],
                out1_hbm.at[pl.ds(base + off, sz)], semi.at[0]).wait())

    return _edge


def kernel(h, pos, edge_index, style, W1, b1, W2, b2, Wf, bf, Wg1, bg1,
           Wg2, bg2, Ws, bs):
    n, c = h.shape
    e = edge_index.shape[1]
    csty = style.shape[1]
    blk = 5000
    grid = n // blk

    src = edge_index[0]
    dst = edge_index[1]
    posp = jnp.pad(pos, ((0, 0), (0, c - pos.shape[1])))
    W2p = jnp.zeros((c, c), jnp.float32).at[:, :pos.shape[1]].set(W2)
    b2p = jnp.zeros((1, c), jnp.float32).at[0, :pos.shape[1]].set(b2)
    Wf3p = jnp.zeros((c, c), jnp.float32).at[:pos.shape[1], :].set(Wf[:pos.shape[1]])
    WfH = Wf[pos.shape[1]:]

    row_spec = pl.BlockSpec((blk, c), lambda i: (i, 0))
    mat_spec = pl.BlockSpec((c, c), lambda i: (0, 0))
    vec_spec = pl.BlockSpec((1, c), lambda i: (0, 0))

    U, V = pl.pallas_call(
        _pre_body,
        grid=(grid,),
        in_specs=[row_spec, row_spec, mat_spec, vec_spec, mat_spec, vec_spec,
                  mat_spec, mat_spec, vec_spec],
        out_specs=[row_spec, row_spec],
        out_shape=[jax.ShapeDtypeStruct((n, c), jnp.float32)] * 2,
    )(h, posp, W1, b1.reshape(1, c), W2p, b2p, Wf3p, WfH, bf.reshape(1, c))

    p0, p1 = _make_edge_kernel(n, c, e)(src, dst, U, V)

    def ph0_row(ph, i):
        return (jnp.where(ph == 0, i, grid - 1), 0)

    def ph1_row(ph, i):
        return (jnp.where(ph == 0, 0, i), 0)

    const2 = lambda ph, i: (0, 0)
    out = pl.pallas_call(
        _make_post_body(n, c, blk),
        grid=(2, grid),
        in_specs=[pl.BlockSpec((blk, c), ph0_row),
                  pl.BlockSpec((blk, c), ph0_row),
                  pl.BlockSpec((blk, c), ph0_row),
                  pl.BlockSpec((c, c), const2),
                  pl.BlockSpec((1, c), const2),
                  pl.BlockSpec((c, c), const2),
                  pl.BlockSpec((1, c), const2),
                  pl.BlockSpec((blk, csty), ph1_row),
                  pl.BlockSpec((csty, 2 * c), const2),
                  pl.BlockSpec((1, 2 * c), const2)],
        out_specs=pl.BlockSpec((blk, c), ph1_row),
        out_shape=jax.ShapeDtypeStruct((n, c), jnp.float32),
        scratch_shapes=[pltpu.VMEM((n, c), jnp.float32),
                        pltpu.VMEM((1, c), jnp.float32),
                        pltpu.VMEM((1, c), jnp.float32)],
    )(p0, p1, h, Wg1, bg1.reshape(1, c), Wg2,
      bg2.reshape(1, c), style, Ws, bs.reshape(1, 2 * c))
    return out


# speedup vs baseline: 17.2319x; 1.0214x over previous
import functools

import jax
import jax.numpy as jnp
from jax import lax
from jax.experimental import pallas as pl
from jax.experimental.pallas import tpu as pltpu
from jax.experimental.pallas import tpu_sc as plsc

_NC = 2
_NS = 16
_NW = _NC * _NS
_CHUNK = 32
_LANES = 16


def _pre_body(h_ref, posp_ref, W1_ref, b1_ref, W2p_ref, b2p_ref, Wf3p_ref,
              WfH_ref, bf_ref, U_ref, V_ref):
    h = h_ref[...]
    posp = posp_ref[...]
    t = jnp.maximum(
        jnp.dot(h, W1_ref[...], preferred_element_type=jnp.float32) + b1_ref[...], 0.0)
    deltap = jnp.tanh(
        jnp.dot(t, W2p_ref[...], preferred_element_type=jnp.float32) + b2p_ref[...])
    u = (jnp.dot(h, WfH_ref[...], preferred_element_type=jnp.float32)
         + jnp.dot(posp, Wf3p_ref[...], preferred_element_type=jnp.float32))
    v = (jnp.dot(deltap - posp, Wf3p_ref[...], preferred_element_type=jnp.float32)
         + bf_ref[...])
    U_ref[...] = u
    V_ref[...] = v


def _make_post_body(n, c, blk):
    def _post_body(p0_ref, p1_ref, h_ref, Wg1_ref, bg1_ref,
                   Wg2_ref, bg2_ref, style_ref, Ws_ref, bs_ref, o_ref,
                   h2_vmem, sum_ref, sumsq_ref):
        ph = pl.program_id(0)
        i = pl.program_id(1)
        row0 = pl.multiple_of(i * blk, 8)

        @pl.when(ph == 0)
        def _():
            aggr = p0_ref[...] + p1_ref[...]
            t = jnp.maximum(
                jnp.dot(aggr, Wg1_ref[...],
                        preferred_element_type=jnp.float32) + bg1_ref[...], 0.0)
            out = jnp.dot(t, Wg2_ref[...],
                          preferred_element_type=jnp.float32) + bg2_ref[...]
            h2 = h_ref[...] + out
            h2 = jnp.where(h2 >= 0, h2, 0.2 * h2)
            h2_vmem[pl.ds(row0, blk), :] = h2
            ps = jnp.sum(h2, axis=0, keepdims=True)
            pss = jnp.sum(h2 * h2, axis=0, keepdims=True)

            @pl.when(i == 0)
            def _():
                sum_ref[...] = ps
                sumsq_ref[...] = pss

            @pl.when(i != 0)
            def _():
                sum_ref[...] += ps
                sumsq_ref[...] += pss

        @pl.when(ph == 1)
        def _():
            mean = sum_ref[...] / n
            var = sumsq_ref[...] / n - mean * mean
            rstd = lax.rsqrt(var + 1e-5)
            s = jnp.dot(style_ref[...], Ws_ref[...],
                        preferred_element_type=jnp.float32) + bs_ref[...]
            gamma = s[:, :c]
            beta = s[:, c:]
            h2 = h2_vmem[pl.ds(row0, blk), :]
            o_ref[...] = gamma * ((h2 - mean) * rstd) + beta
    return _post_body


def _chunk_sizes(total):
    sizes = [_CHUNK] * (total // _CHUNK)
    if total % _CHUNK:
        sizes.append(total % _CHUNK)
    return sizes


def _make_edge_kernel(n, c, e):
    rows_main = ((n // 8) // _NS) * 8
    rows_last = n - (_NS - 1) * rows_main
    nchunks = e // _CHUNK
    nworkers = _NW
    iters = (nchunks + nworkers - 1) // nworkers
    nvec = c // _LANES
    mesh = plsc.VectorSubcoreMesh(core_axis_name="c", subcore_axis_name="s")

    @functools.partial(
        pl.kernel,
        out_type=[jax.ShapeDtypeStruct((n, c), jnp.float32),
                  jax.ShapeDtypeStruct((n, c), jnp.float32)],
        mesh=mesh,
        scratch_types=[
            pltpu.VMEM((8, _CHUNK), jnp.int32),
            pltpu.VMEM((8, _CHUNK), jnp.int32),
            pltpu.VMEM((4, _CHUNK, c), jnp.float32),
            pltpu.VMEM((4, _CHUNK, c), jnp.float32),
            pltpu.VMEM((2, _CHUNK, c), jnp.float32),
            pltpu.VMEM_SHARED((n, c), jnp.float32),
            pltpu.SemaphoreType.DMA((8,)),
            pltpu.SemaphoreType.DMA((4,)),
            pltpu.SemaphoreType.DMA((2,)),
        ],
    )
    def _edge(src_hbm, dst_hbm, u_hbm, v_hbm, out0_hbm, out1_hbm,
              sidx, didx, u_v, v_v, m_v, aggr_sh, semi, semg, semsc):
        ci = lax.axis_index("c")
        si = lax.axis_index("s")
        w = si * _NC + ci

        def cid_of(t):
            return w + t * nworkers

        def issue_idx(t, ts):
            q = ts % 8

            @pl.when(cid_of(t) < nchunks)
            def _():
                off = cid_of(t) * _CHUNK
                pltpu.async_copy(src_hbm.at[pl.ds(off, _CHUNK)], sidx.at[q],
                                 semi.at[q])
                pltpu.async_copy(dst_hbm.at[pl.ds(off, _CHUNK)], didx.at[q],
                                 semi.at[q])

        def issue_gathers(t, ts):
            q, b = ts % 8, ts % 4

            @pl.when(cid_of(t) < nchunks)
            def _():
                off = cid_of(t) * _CHUNK
                pltpu.make_async_copy(src_hbm.at[pl.ds(off, _CHUNK)],
                                      sidx.at[q], semi.at[q]).wait()
                pltpu.make_async_copy(dst_hbm.at[pl.ds(off, _CHUNK)],
                                      didx.at[q], semi.at[q]).wait()
                pltpu.async_copy(u_hbm.at[sidx.at[q]], u_v.at[b], semg.at[b])
                pltpu.async_copy(v_hbm.at[didx.at[q]], v_v.at[b], semg.at[b])

        def wait_gathers(t, ts):
            q, b = ts % 8, ts % 4

            @pl.when(cid_of(t) < nchunks)
            def _():
                pltpu.make_async_copy(u_hbm.at[sidx.at[q]], u_v.at[b],
                                      semg.at[b]).wait()
                pltpu.make_async_copy(v_hbm.at[didx.at[q]], v_v.at[b],
                                      semg.at[b]).wait()

        def wait_scatter(ts):
            q, b = ts % 8, ts % 2
            pltpu.make_async_copy(m_v.at[b], aggr_sh.at[didx.at[q]],
                                  semsc.at[b]).wait()

        def drain_scatter(t, ts):
            @pl.when(cid_of(t) < nchunks)
            def _():
                wait_scatter(ts)

        def process(t, ts):
            q, b3, b2 = ts % 8, ts % 4, ts % 2

            @pl.when(cid_of(t) < nchunks)
            def _():
                @plsc.parallel_loop(0, _CHUNK, unroll=2)
                def _crow(r):
                    for j in range(nvec):
                        sl = pl.ds(j * _LANES, _LANES)
                        m_v[b2, r, sl] = jnp.maximum(
                            u_v[b3, r, sl] + v_v[b3, r, sl], 0.0)
                pltpu.async_copy(m_v.at[b2], aggr_sh.at[didx.at[q]],
                                 semsc.at[b2], add=True)

        def zrow(r, carry):
            for j in range(nvec):
                m_v[0, r, pl.ds(j * _LANES, _LANES)] = jnp.zeros(
                    (_LANES,), jnp.float32)
            return carry
        lax.fori_loop(0, _CHUNK, zrow, 0)

        base = pl.multiple_of(si * rows_main, 8)

        def _stripe_copy(row_fn):
            @pl.when(si < _NS - 1)
            def _():
                off = 0
                for sz in _chunk_sizes(rows_main):
                    row_fn(off, sz)
                    off += sz

            @pl.when(si == _NS - 1)
            def _():
                off = 0
                for sz in _chunk_sizes(rows_last):
                    row_fn(off, sz)
                    off += sz

        _stripe_copy(lambda off, sz: pltpu.async_copy(
            m_v.at[0, pl.ds(0, sz)], aggr_sh.at[pl.ds(base + off, sz)],
            semi.at[0]))
        _stripe_copy(lambda off, sz: pltpu.make_async_copy(
            m_v.at[0, pl.ds(0, sz)], aggr_sh.at[pl.ds(base + off, sz)],
            semi.at[0]).wait())
        plsc.subcore_barrier()

        for t0 in (0, 1, 2, 3, 4):
            issue_idx(t0, t0)
        issue_gathers(0, 0)
        issue_gathers(1, 1)
        issue_gathers(2, 2)
        for t0 in (0, 1):
            wait_gathers(t0, t0)
            issue_gathers(t0 + 3, t0 + 3)
            process(t0, t0)
            issue_idx(t0 + 5, t0 + 5)

        nocto = (iters + 7) // 8

        def outer(g, carry):
            for ts in (2, 3, 4, 5, 6, 7, 8, 9):
                t = 8 * g + ts
                wait_gathers(t, ts)
                issue_gathers(t + 3, ts + 3)
                drain_scatter(t - 2, ts - 2)
                process(t, ts)
                issue_idx(t + 5, ts + 5)
            return carry
        lax.fori_loop(0, nocto, outer, 0)

        tlast = 2 + 8 * nocto
        drain_scatter(tlast - 2, tlast - 2)
        drain_scatter(tlast - 1, tlast - 1)
        plsc.subcore_barrier()

        @pl.when(ci == 0)
        def _():
            _stripe_copy(lambda off, sz: pltpu.async_copy(
                aggr_sh.at[pl.ds(base + off, sz)],
                out0_hbm.at[pl.ds(base + off, sz)], semi.at[0]))
            _stripe_copy(lambda off, sz: pltpu.make_async_copy(
                aggr_sh.at[pl.ds(base + off, sz)],
                out0_hbm.at[pl.ds(base + off, sz)], semi.at[0]).wait())

        @pl.when(ci == 1)
        def _():
            _stripe_copy(lambda off, sz: pltpu.async_copy(
                aggr_sh.at[pl.ds(base + off, sz)],
                out1_hbm.at[pl.ds(base + off, sz)], semi.at[0]))
            _stripe_copy(lambda off, sz: pltpu.make_async_copy(
                aggr_sh.at[pl.ds(base + off, sz)],
                out1_hbm.at[pl.ds(base + off, sz)], semi.at[0]).wait())

    return _edge


def kernel(h, pos, edge_index, style, W1, b1, W2, b2, Wf, bf, Wg1, bg1,
           Wg2, bg2, Ws, bs):
    n, c = h.shape
    e = edge_index.shape[1]
    csty = style.shape[1]
    blk = 5000
    grid = n // blk

    src = edge_index[0]
    dst = edge_index[1]
    posp = jnp.pad(pos, ((0, 0), (0, c - pos.shape[1])))
    W2p = jnp.zeros((c, c), jnp.float32).at[:, :pos.shape[1]].set(W2)
    b2p = jnp.zeros((1, c), jnp.float32).at[0, :pos.shape[1]].set(b2)
    Wf3p = jnp.zeros((c, c), jnp.float32).at[:pos.shape[1], :].set(Wf[:pos.shape[1]])
    WfH = Wf[pos.shape[1]:]

    row_spec = pl.BlockSpec((blk, c), lambda i: (i, 0))
    mat_spec = pl.BlockSpec((c, c), lambda i: (0, 0))
    vec_spec = pl.BlockSpec((1, c), lambda i: (0, 0))

    U, V = pl.pallas_call(
        _pre_body,
        grid=(grid,),
        in_specs=[row_spec, row_spec, mat_spec, vec_spec, mat_spec, vec_spec,
                  mat_spec, mat_spec, vec_spec],
        out_specs=[row_spec, row_spec],
        out_shape=[jax.ShapeDtypeStruct((n, c), jnp.float32)] * 2,
    )(h, posp, W1, b1.reshape(1, c), W2p, b2p, Wf3p, WfH, bf.reshape(1, c))

    p0, p1 = _make_edge_kernel(n, c, e)(src, dst, U, V)

    def ph0_row(ph, i):
        return (jnp.where(ph == 0, i, grid - 1), 0)

    def ph1_row(ph, i):
        return (jnp.where(ph == 0, 0, i), 0)

    const2 = lambda ph, i: (0, 0)
    out = pl.pallas_call(
        _make_post_body(n, c, blk),
        grid=(2, grid),
        in_specs=[pl.BlockSpec((blk, c), ph0_row),
                  pl.BlockSpec((blk, c), ph0_row),
                  pl.BlockSpec((blk, c), ph0_row),
                  pl.BlockSpec((c, c), const2),
                  pl.BlockSpec((1, c), const2),
                  pl.BlockSpec((c, c), const2),
                  pl.BlockSpec((1, c), const2),
                  pl.BlockSpec((blk, csty), ph1_row),
                  pl.BlockSpec((csty, 2 * c), const2),
                  pl.BlockSpec((1, 2 * c), const2)],
        out_specs=pl.BlockSpec((blk, c), ph1_row),
        out_shape=jax.ShapeDtypeStruct((n, c), jnp.float32),
        scratch_shapes=[pltpu.VMEM((n, c), jnp.float32),
                        pltpu.VMEM((1, c), jnp.float32),
                        pltpu.VMEM((1, c), jnp.float32)],
    )(p0, p1, h, Wg1, bg1.reshape(1, c), Wg2,
      bg2.reshape(1, c), style, Ws, bs.reshape(1, 2 * c))
    return out
